# Initial kernel scaffold; baseline (speedup 1.0000x reference)
import functools

import jax
import jax.numpy as jnp
from jax import lax
from jax.experimental import pallas as pl
from jax.experimental.pallas import tpu as pltpu
from jax.experimental.pallas import tpu_sc as plsc

B_TOK = 16384
T_TOK = 50
D = 64
N = B_TOK * T_TOK
NC = 2
NS = 16
NW = NC * NS
PER_W = N // NW
SUB = 128
CHUNK = 1024
N_SUB = CHUNK // SUB
N_OUTER = PER_W // CHUNK
ROWS_PER_W = PER_W // SUB

_mesh = plsc.VectorSubcoreMesh(core_axis_name="c", subcore_axis_name="s")


@functools.partial(
    pl.kernel,
    out_type=jax.ShapeDtypeStruct((N, D), jnp.float32),
    mesh=_mesh,
    scratch_types=[
        pltpu.VMEM((N_SUB, SUB), jnp.int32),
        pltpu.VMEM((CHUNK, D), jnp.float32),
        pltpu.SemaphoreType.DMA,
    ],
)
def _sc_gather(idx_hbm, table_hbm, out_hbm, idx_v, rows_v, gsem):
    wid = lax.axis_index("s") * NC + lax.axis_index("c")
    row_base = wid * ROWS_PER_W
    out_base = wid * PER_W

    @pl.loop(0, N_OUTER)
    def _outer(i):
        pltpu.sync_copy(idx_hbm.at[pl.ds(row_base + i * N_SUB, N_SUB), :], idx_v)
        copies = [
            pltpu.async_copy(
                table_hbm.at[idx_v.at[j]],
                rows_v.at[pl.ds(j * SUB, SUB), :],
                gsem,
            )
            for j in range(N_SUB)
        ]
        for c in copies:
            c.wait()
        pltpu.sync_copy(rows_v, out_hbm.at[pl.ds(out_base + i * CHUNK, CHUNK), :])


def kernel(token_idx, tokenvectors):
    idx2 = token_idx.reshape(N // SUB, SUB).astype(jnp.int32)
    out = _sc_gather(idx2, tokenvectors)
    return out.reshape(B_TOK, T_TOK, D)


# speedup vs baseline: 1.8448x; 1.8448x over previous
import functools

import jax
import jax.numpy as jnp
from jax import lax
from jax.experimental import pallas as pl
from jax.experimental.pallas import tpu as pltpu
from jax.experimental.pallas import tpu_sc as plsc

B_TOK = 16384
T_TOK = 50
D = 64
N = B_TOK * T_TOK
NC = 2
NS = 16
NW = NC * NS
PER_W = N // NW
SUB = 128
CHUNK = 1024
N_SUB = CHUNK // SUB
N_OUTER = PER_W // CHUNK
ROWS_PER_W = PER_W // SUB

_mesh = plsc.VectorSubcoreMesh(core_axis_name="c", subcore_axis_name="s")


@functools.partial(
    pl.kernel,
    out_type=jax.ShapeDtypeStruct((N, D), jnp.float32),
    mesh=_mesh,
    scratch_types=[
        pltpu.VMEM((N_SUB, SUB), jnp.int32),
        pltpu.VMEM((CHUNK, D), jnp.float32),
        pltpu.SemaphoreType.DMA,
    ],
    compiler_params=pltpu.CompilerParams(use_tc_tiling_on_sc=False),
)
def _sc_gather(idx_hbm, table_hbm, out_hbm, idx_v, rows_v, gsem):
    wid = lax.axis_index("s") * NC + lax.axis_index("c")
    row_base = wid * ROWS_PER_W
    out_base = wid * PER_W

    @pl.loop(0, N_OUTER)
    def _outer(i):
        pltpu.sync_copy(idx_hbm.at[pl.ds(row_base + i * N_SUB, N_SUB), :], idx_v)
        copies = [
            pltpu.async_copy(
                table_hbm.at[idx_v.at[j]],
                rows_v.at[pl.ds(j * SUB, SUB), :],
                gsem,
            )
            for j in range(N_SUB)
        ]
        for c in copies:
            c.wait()
        pltpu.sync_copy(rows_v, out_hbm.at[pl.ds(out_base + i * CHUNK, CHUNK), :])


def kernel(token_idx, tokenvectors):
    idx2 = token_idx.reshape(N // SUB, SUB).astype(jnp.int32)
    out = _sc_gather(idx2, tokenvectors)
    return out.reshape(B_TOK, T_TOK, D)


# speedup vs baseline: 1.8733x; 1.0155x over previous
import functools

import jax
import jax.numpy as jnp
from jax import lax
from jax.experimental import pallas as pl
from jax.experimental.pallas import tpu as pltpu
from jax.experimental.pallas import tpu_sc as plsc

B_TOK = 16384
T_TOK = 50
D = 64
N = B_TOK * T_TOK
NC = 2
NS = 16
NW = NC * NS
PER_W = N // NW
SUB = 128
SUBS = 5
CHUNK = SUB * SUBS
N_OUTER = PER_W // CHUNK
ROWS_PER_W = PER_W // SUB

_mesh = plsc.VectorSubcoreMesh(core_axis_name="c", subcore_axis_name="s")


@functools.partial(
    pl.kernel,
    out_type=jax.ShapeDtypeStruct((N, D), jnp.float32),
    mesh=_mesh,
    scratch_types=[
        pltpu.VMEM((ROWS_PER_W, SUB), jnp.int32),
        pltpu.VMEM((CHUNK, D), jnp.float32),
        pltpu.VMEM((CHUNK, D), jnp.float32),
        pltpu.SemaphoreType.DMA,
        pltpu.SemaphoreType.DMA,
        pltpu.SemaphoreType.DMA,
        pltpu.SemaphoreType.DMA,
    ],
    compiler_params=pltpu.CompilerParams(use_tc_tiling_on_sc=False),
)
def _sc_gather(idx_hbm, table_hbm, out_hbm, idx_v, rows0, rows1,
               gsem0, gsem1, osem0, osem1):
    wid = lax.axis_index("s") * NC + lax.axis_index("c")
    out_base = wid * PER_W

    pltpu.sync_copy(idx_hbm.at[pl.ds(wid * ROWS_PER_W, ROWS_PER_W), :], idx_v)

    def fire_gathers(k, rows, gsem):
        for j in range(SUBS):
            pltpu.async_copy(
                table_hbm.at[idx_v.at[k * SUBS + j]],
                rows.at[pl.ds(j * SUB, SUB), :],
                gsem,
            )

    def drain_gather(rows, gsem):
        pltpu.make_async_copy(out_hbm.at[pl.ds(0, CHUNK), :], rows, gsem).wait()

    def drain_writeout(rows, osem):
        pltpu.make_async_copy(rows, out_hbm.at[pl.ds(0, CHUNK), :], osem).wait()

    fire_gathers(0, rows0, gsem0)

    @pl.loop(0, N_OUTER, step=2)
    def _outer(i):
        for half in range(2):
            k = i + half
            if half == 0:
                rows_cur, rows_nxt = rows0, rows1
                gsem_cur, gsem_nxt = gsem0, gsem1
                osem_cur, osem_nxt = osem0, osem1
            else:
                rows_cur, rows_nxt = rows1, rows0
                gsem_cur, gsem_nxt = gsem1, gsem0
                osem_cur, osem_nxt = osem1, osem0

            @pl.when(k > 0)
            def _():
                drain_writeout(rows_nxt, osem_nxt)

            @pl.when(k + 1 < N_OUTER)
            def _():
                fire_gathers(k + 1, rows_nxt, gsem_nxt)

            drain_gather(rows_cur, gsem_cur)
            pltpu.async_copy(
                rows_cur,
                out_hbm.at[pl.ds(out_base + k * CHUNK, CHUNK), :],
                osem_cur,
            )

    drain_writeout(rows1, osem1)


def kernel(token_idx, tokenvectors):
    idx2 = token_idx.reshape(N // SUB, SUB).astype(jnp.int32)
    out = _sc_gather(idx2, tokenvectors)
    return out.reshape(B_TOK, T_TOK, D)


# speedup vs baseline: 1.8856x; 1.0065x over previous
import functools

import jax
import jax.numpy as jnp
from jax import lax
from jax.experimental import pallas as pl
from jax.experimental.pallas import tpu as pltpu
from jax.experimental.pallas import tpu_sc as plsc

B_TOK = 16384
T_TOK = 50
D = 64
NC = 2
NS = 16
NW = NC * NS
ROWS_W = B_TOK // NW
RB = 8
N_OUTER = ROWS_W // RB

_mesh = plsc.VectorSubcoreMesh(core_axis_name="c", subcore_axis_name="s")


@functools.partial(
    pl.kernel,
    out_type=jax.ShapeDtypeStruct((B_TOK, T_TOK, D), jnp.float32),
    mesh=_mesh,
    scratch_types=[
        pltpu.VMEM((ROWS_W, T_TOK), jnp.int32),
        pltpu.VMEM((RB, T_TOK, D), jnp.float32),
        pltpu.VMEM((RB, T_TOK, D), jnp.float32),
        pltpu.SemaphoreType.DMA,
        pltpu.SemaphoreType.DMA,
        pltpu.SemaphoreType.DMA,
        pltpu.SemaphoreType.DMA,
    ],
    compiler_params=pltpu.CompilerParams(use_tc_tiling_on_sc=False),
)
def _sc_gather(idx_hbm, table_hbm, out_hbm, idx_v, rows0, rows1,
               gsem0, gsem1, osem0, osem1):
    wid = lax.axis_index("s") * NC + lax.axis_index("c")
    row_base = wid * ROWS_W

    pltpu.sync_copy(idx_hbm.at[pl.ds(row_base, ROWS_W), :], idx_v)

    def fire_gathers(k, rows, gsem):
        for r in range(RB):
            pltpu.async_copy(
                table_hbm.at[idx_v.at[k * RB + r]],
                rows.at[r],
                gsem,
            )

    def drain_gather(rows, gsem):
        pltpu.make_async_copy(
            out_hbm.at[pl.ds(0, RB)], rows, gsem).wait()

    def drain_writeout(rows, osem):
        pltpu.make_async_copy(
            rows, out_hbm.at[pl.ds(0, RB)], osem).wait()

    fire_gathers(0, rows0, gsem0)

    @pl.loop(0, N_OUTER, step=2)
    def _outer(i):
        for half in range(2):
            k = i + half
            if half == 0:
                rows_cur, rows_nxt = rows0, rows1
                gsem_cur, gsem_nxt = gsem0, gsem1
                osem_cur, osem_nxt = osem0, osem1
            else:
                rows_cur, rows_nxt = rows1, rows0
                gsem_cur, gsem_nxt = gsem1, gsem0
                osem_cur, osem_nxt = osem1, osem0

            @pl.when(k > 0)
            def _():
                drain_writeout(rows_nxt, osem_nxt)

            @pl.when(k + 1 < N_OUTER)
            def _():
                fire_gathers(k + 1, rows_nxt, gsem_nxt)

            drain_gather(rows_cur, gsem_cur)
            pltpu.async_copy(
                rows_cur,
                out_hbm.at[pl.ds(row_base + k * RB, RB)],
                osem_cur,
            )

    drain_writeout(rows1, osem1)


def kernel(token_idx, tokenvectors):
    return _sc_gather(token_idx, tokenvectors)
